# baseline (speedup 1.0000x reference)
import functools

import jax
import jax.numpy as jnp
from jax import lax
from jax.experimental import pallas as pl
from jax.experimental.pallas import tpu as pltpu
from jax.experimental.pallas import tpu_sc as plsc

_NC = 2
_NS = 16
_CW = 64
_EB = 128


def _sc_agg_body(hc_hbm, src_hbm, addr_hbm, z_hbm, *rest, nchunk, nb,
                 rows_s, with_deg):
    if with_deg:
        (s_out, deg_out, src_v, addr_v, r0, r1, r2, r3, s_sh,
         sg0, sg1, sg2, sg3, ss0, ss1, ss2, ss3) = rest
    else:
        (s_out, src_v, addr_v, r0, r1, r2, r3, s_sh,
         sg0, sg1, sg2, sg3, ss0, ss1, ss2, ss3) = rest
        deg_out = None
    bufs = (r0, r1, r2, r3)
    sgs = (sg0, sg1, sg2, sg3)
    sss = (ss0, ss1, ss2, ss3)
    cid = lax.axis_index("c")
    sid = lax.axis_index("s")
    wid = cid * _NS + sid
    rpt = rows_s // _NS
    row0 = sid * rpt

    pltpu.sync_copy(src_hbm.at[wid], src_v)
    pltpu.sync_copy(addr_hbm.at[wid], addr_v)

    def _zero_own():
        pltpu.sync_copy(z_hbm.at[pl.ds(row0, rpt)],
                        s_sh.at[pl.ds(row0, rpt)])

    for c in range(nchunk):
        _zero_own()
        plsc.subcore_barrier()

        hcc = hc_hbm.at[c]

        def _fire_g(j, slot):
            pltpu.async_copy(hcc.at[src_v.at[j]], bufs[slot], sgs[slot])

        def _drain_g(j, slot):
            pltpu.make_async_copy(hcc.at[src_v.at[j]], bufs[slot],
                                  sgs[slot]).wait()

        def _fire_s(j, slot):
            pltpu.async_copy(bufs[slot], s_sh.at[addr_v.at[j]], sss[slot],
                             add=True)

        def _drain_s(j, slot):
            pltpu.make_async_copy(bufs[slot], s_sh.at[addr_v.at[j]],
                                  sss[slot]).wait()

        _fire_g(0, 0)
        _fire_g(1, 1)

        def _pipe(g, _):
            for u in range(4):
                j = 4 * g + u
                _drain_g(j, u)
                _fire_s(j, u)

                @pl.when(j + 2 < nb)
                def _():
                    @pl.when(j >= 2)
                    def _():
                        _drain_s(j - 2, (u + 2) % 4)
                    _fire_g(j + 2, (u + 2) % 4)
            return 0

        lax.fori_loop(0, nb // 4, _pipe, 0)
        for k in range(nb - 4, nb):
            _drain_s(k, k % 4)
        plsc.subcore_barrier()

        pltpu.sync_copy(s_sh.at[pl.ds(row0, rpt)],
                        s_out.at[cid, c, pl.ds(row0, rpt)])
        plsc.subcore_barrier()

    if with_deg:
        _zero_own()

        def _fill(i, _):
            for k in range(_CW // 16):
                r0[i, pl.ds(16 * k, 16)] = jnp.ones((16,), jnp.float32)
            return 0
        lax.fori_loop(0, _EB, _fill, 0)
        plsc.subcore_barrier()

        def _deg_fire(j, _):
            pltpu.async_copy(r0, s_sh.at[addr_v.at[j]], ss0, add=True)
            return 0
        lax.fori_loop(0, nb, _deg_fire, 0)

        def _deg_drain(j, _):
            pltpu.make_async_copy(r0, s_sh.at[addr_v.at[j]], ss0).wait()
            return 0
        lax.fori_loop(0, nb, _deg_drain, 0)
        plsc.subcore_barrier()
        pltpu.sync_copy(s_sh.at[pl.ds(row0, rpt)],
                        deg_out.at[cid, pl.ds(row0, rpt)])


def _sc_aggregate(hc, srcb, addrb, z_hbm, with_deg):
    nchunk, _, _ = hc.shape
    nb = srcb.shape[1]
    rows_s = z_hbm.shape[0]
    mesh = plsc.VectorSubcoreMesh(core_axis_name="c", subcore_axis_name="s",
                                  num_cores=_NC, num_subcores=_NS)
    out_type = [jax.ShapeDtypeStruct((_NC, nchunk, rows_s, _CW), jnp.float32)]
    scratch = [
        pltpu.VMEM((nb, _EB), jnp.int32),
        pltpu.VMEM((nb, _EB), jnp.int32),
    ] + [pltpu.VMEM((_EB, _CW), jnp.float32)] * 4 + [
        pltpu.VMEM_SHARED((rows_s, _CW), jnp.float32),
    ] + [pltpu.SemaphoreType.DMA] * 8
    inputs = [hc, srcb, addrb, z_hbm]
    if with_deg:
        out_type.append(
            jax.ShapeDtypeStruct((_NC, rows_s, _CW), jnp.float32))
    body = functools.partial(_sc_agg_body, nchunk=nchunk, nb=nb,
                             rows_s=rows_s, with_deg=with_deg)
    return pl.kernel(
        body, out_type=out_type, mesh=mesh, scratch_types=scratch,
        compiler_params=pltpu.CompilerParams(use_tc_tiling_on_sc=False),
    )(*inputs)


def _layer_body(h_ref, s_ref, deg_ref, wroot_ref, wrel_ref, b_ref, out_ref,
                *, relu, nchunk):
    h = h_ref[...]
    dsum = deg_ref[0] + deg_ref[1]
    n0 = 1.0 / jnp.clip(dsum[0][:, 0:1], 1.0, None)
    n1 = 1.0 / jnp.clip(dsum[1][:, 0:1], 1.0, None)
    s0 = jnp.concatenate(
        [s_ref[0, c, 0] + s_ref[1, c, 0] for c in range(nchunk)], axis=1)
    s1 = jnp.concatenate(
        [s_ref[0, c, 1] + s_ref[1, c, 1] for c in range(nchunk)], axis=1)
    acc = jnp.dot(h, wroot_ref[...], preferred_element_type=jnp.float32)
    acc = acc + jnp.dot(s0 * n0, wrel_ref[0],
                        preferred_element_type=jnp.float32)
    acc = acc + jnp.dot(s1 * n1, wrel_ref[1],
                        preferred_element_type=jnp.float32)
    acc = acc + b_ref[...]
    out_ref[...] = jnp.maximum(acc, 0.0) if relu else acc


def _rgcn_dense(h, S5, deg4, W_root, W_rel, b, relu):
    n, f = h.shape
    o = W_rel.shape[2]
    nchunk = S5.shape[1]
    npr = S5.shape[3]
    bn = 1000
    grid = n // bn
    return pl.pallas_call(
        functools.partial(_layer_body, relu=relu, nchunk=nchunk),
        grid=(grid,),
        in_specs=[
            pl.BlockSpec((bn, f), lambda i: (i, 0)),
            pl.BlockSpec((2, nchunk, 2, bn, _CW), lambda i: (0, 0, 0, i, 0)),
            pl.BlockSpec((2, 2, bn, _CW), lambda i: (0, 0, i, 0)),
            pl.BlockSpec((f, o), lambda i: (0, 0)),
            pl.BlockSpec((2, f, o), lambda i: (0, 0, 0)),
            pl.BlockSpec((1, o), lambda i: (0, 0)),
        ],
        out_specs=pl.BlockSpec((bn, o), lambda i: (i, 0)),
        out_shape=jax.ShapeDtypeStruct((n, o), jnp.float32),
    )(h, S5, deg4, W_root, W_rel, b.reshape(1, o))


def _lstm_body(msg_ref, emb_ref, wih_ref, whh_ref, bg_ref, wfc_ref, bfc_ref,
               out_ref, xg_ref, *, L, B, HID, VOCAB):
    m = msg_ref[...]
    mx = jnp.max(m, axis=1, keepdims=True)
    col = lax.broadcasted_iota(jnp.int32, m.shape, 1)
    tok = jnp.min(jnp.where(m == mx, col, VOCAB), axis=1, keepdims=True)
    onehot = (col == tok).astype(jnp.float32)
    xe = jnp.dot(onehot, emb_ref[...], preferred_element_type=jnp.float32)
    xg = jnp.dot(xe, wih_ref[...], preferred_element_type=jnp.float32)
    xg = xg + bg_ref[...]
    xg_ref[...] = xg.reshape(L, B, 4 * HID)

    def step(l, carry):
        h, c = carry
        gates = xg_ref[l] + jnp.dot(h, whh_ref[...],
                                    preferred_element_type=jnp.float32)
        i = jax.nn.sigmoid(gates[:, :HID])
        f = jax.nn.sigmoid(gates[:, HID:2 * HID])
        g = jnp.tanh(gates[:, 2 * HID:3 * HID])
        o = jax.nn.sigmoid(gates[:, 3 * HID:])
        c = f * c + i * g
        h = o * jnp.tanh(c)
        return (h, c)

    h0 = jnp.zeros((B, HID), jnp.float32)
    hT, _ = lax.fori_loop(0, L, step, (h0, h0))
    out_ref[...] = jnp.dot(hT, wfc_ref[...],
                           preferred_element_type=jnp.float32) + bfc_ref[...]


def _msg_branch(message, emb_table, W_ih, W_hh, b_ih, b_hh, W_fc, b_fc):
    B, L, VOCAB = message.shape
    HID = W_hh.shape[1]
    EMB = emb_table.shape[1]
    msg_tm2d = jnp.swapaxes(message, 0, 1).reshape(L * B, VOCAB)
    bg = (b_ih + b_hh).reshape(1, 4 * HID)
    return pl.pallas_call(
        functools.partial(_lstm_body, L=L, B=B, HID=HID, VOCAB=VOCAB),
        out_shape=jax.ShapeDtypeStruct((B, EMB), jnp.float32),
        scratch_shapes=[pltpu.VMEM((L, B, 4 * HID), jnp.float32)],
    )(msg_tm2d, emb_table, W_ih.T, W_hh.T, bg, W_fc.T, b_fc.reshape(1, EMB))


def _logits_body(hid_ref, h_ref, out_ref):
    lg = lax.dot_general(hid_ref[...], h_ref[...], (((1,), (1,)), ((), ())),
                         preferred_element_type=jnp.float32)
    mx = jnp.max(lg, axis=1, keepdims=True)
    lse = jnp.log(jnp.sum(jnp.exp(lg - mx), axis=1, keepdims=True)) + mx
    out_ref[...] = lg - lse


def _logits(hidden, h):
    B = hidden.shape[0]
    n = h.shape[0]
    return pl.pallas_call(
        _logits_body,
        out_shape=jax.ShapeDtypeStruct((B, n), jnp.float32),
    )(hidden, h)


def _chunked(h):
    n, f = h.shape
    return jnp.transpose(h.reshape(n, f // _CW, _CW), (1, 0, 2))


def kernel(message, x, node_feats, edge_index, edge_type, nest_tensor,
           W_root1, W_rel1, b1, W_root2, W_rel2, b2, emb_table,
           W_ih, W_hh, b_ih, b_hh, W_fc, b_fc):
    src = edge_index[0]
    dst = edge_index[1]
    n = node_feats.shape[0]
    e = src.shape[0]
    nchunk = node_feats.shape[1] // _CW

    npr = -(-(n + 1) // 64) * 64
    rows_s = 2 * npr
    nb = -(-e // (_NC * _NS * _EB))
    epad = _NC * _NS * nb * _EB
    pad = epad - e
    srcb = jnp.concatenate(
        [src, jnp.zeros((pad,), jnp.int32)]).reshape(_NC * _NS, nb, _EB)
    addr = edge_type * npr + dst
    addrb = jnp.concatenate(
        [addr, jnp.full((pad,), n, jnp.int32)]).reshape(_NC * _NS, nb, _EB)
    z_hbm = jnp.zeros((rows_s, _CW), jnp.float32)

    S1, deg = _sc_aggregate(_chunked(node_feats), srcb, addrb, z_hbm,
                            with_deg=True)
    S1 = S1.reshape(_NC, nchunk, 2, npr, _CW)
    deg4 = deg.reshape(_NC, 2, npr, _CW)
    h1 = _rgcn_dense(node_feats, S1, deg4, W_root1, W_rel1, b1, relu=True)
    (S2,) = _sc_aggregate(_chunked(h1), srcb, addrb, z_hbm, with_deg=False)
    S2 = S2.reshape(_NC, nchunk, 2, npr, _CW)
    h = _rgcn_dense(h1, S2, deg4, W_root2, W_rel2, b2, relu=False)

    hidden = _msg_branch(message, emb_table, W_ih, W_hh, b_ih, b_hh,
                         W_fc, b_fc)
    return _logits(hidden, h)


# speedup vs baseline: 1.0432x; 1.0432x over previous
import functools

import jax
import jax.numpy as jnp
from jax import lax
from jax.experimental import pallas as pl
from jax.experimental.pallas import tpu as pltpu
from jax.experimental.pallas import tpu_sc as plsc

_NC = 2
_NS = 16
_CW = 64
_EB = 128


def _sc_agg_body(hc_hbm, src_hbm, addr_hbm, z_hbm, *rest, nchunk, nb0, nb1,
                 rows_s, with_deg):
    if with_deg:
        (s_out, deg_out, src_v, addr_v, r0, r1, r2, r3, s_sh,
         sg0, sg1, sg2, sg3, ss0, ss1, ss2, ss3) = rest
    else:
        (s_out, src_v, addr_v, r0, r1, r2, r3, s_sh,
         sg0, sg1, sg2, sg3, ss0, ss1, ss2, ss3) = rest
        deg_out = None
    bufs = (r0, r1, r2, r3)
    sgs = (sg0, sg1, sg2, sg3)
    sss = (ss0, ss1, ss2, ss3)
    cid = lax.axis_index("c")
    sid = lax.axis_index("s")
    rpt = rows_s // _NS
    row0 = sid * rpt

    @pl.when(cid == 0)
    def _():
        pltpu.sync_copy(src_hbm.at[pl.ds(sid * nb0, nb0)], src_v)
        pltpu.sync_copy(addr_hbm.at[pl.ds(sid * nb0, nb0)], addr_v)

    @pl.when(cid == 1)
    def _():
        base = _NS * nb0 + sid * nb1
        pltpu.sync_copy(src_hbm.at[pl.ds(base, nb1)],
                        src_v.at[pl.ds(0, nb1)])
        pltpu.sync_copy(addr_hbm.at[pl.ds(base, nb1)],
                        addr_v.at[pl.ds(0, nb1)])

    def _zero_own():
        pltpu.sync_copy(z_hbm.at[pl.ds(row0, rpt)],
                        s_sh.at[pl.ds(row0, rpt)])

    for c in range(nchunk):
        _zero_own()
        plsc.subcore_barrier()

        hcc = hc_hbm.at[c]

        def _fire_g(j, slot):
            pltpu.async_copy(hcc.at[src_v.at[j]], bufs[slot], sgs[slot])

        def _drain_g(j, slot):
            pltpu.make_async_copy(hcc.at[src_v.at[j]], bufs[slot],
                                  sgs[slot]).wait()

        def _fire_s(j, slot):
            pltpu.async_copy(bufs[slot], s_sh.at[addr_v.at[j]], sss[slot],
                             add=True)

        def _drain_s(j, slot):
            pltpu.make_async_copy(bufs[slot], s_sh.at[addr_v.at[j]],
                                  sss[slot]).wait()

        def _pipeline(nb):
            _fire_g(0, 0)
            _fire_g(1, 1)

            def _pipe(g, _):
                for u in range(4):
                    j = 4 * g + u
                    _drain_g(j, u)
                    _fire_s(j, u)

                    @pl.when(j + 2 < nb)
                    def _():
                        @pl.when(j >= 2)
                        def _():
                            _drain_s(j - 2, (u + 2) % 4)
                        _fire_g(j + 2, (u + 2) % 4)
                return 0

            lax.fori_loop(0, nb // 4, _pipe, 0)
            for k in range(nb - 4, nb):
                _drain_s(k, k % 4)

        @pl.when(cid == 0)
        def _():
            _pipeline(nb0)

        @pl.when(cid == 1)
        def _():
            _pipeline(nb1)

        plsc.subcore_barrier()

        pltpu.sync_copy(s_sh.at[pl.ds(row0, rpt)],
                        s_out.at[cid, c, pl.ds(row0, rpt)])
        plsc.subcore_barrier()

    if with_deg:
        _zero_own()

        def _fill(i, _):
            for k in range(_CW // 16):
                r0[i, pl.ds(16 * k, 16)] = jnp.ones((16,), jnp.float32)
            return 0
        lax.fori_loop(0, _EB, _fill, 0)
        plsc.subcore_barrier()

        def _deg_run(nb):
            def _deg_fire(j, _):
                pltpu.async_copy(r0, s_sh.at[addr_v.at[j]], ss0, add=True)
                return 0
            lax.fori_loop(0, nb, _deg_fire, 0)

            def _deg_drain(j, _):
                pltpu.make_async_copy(r0, s_sh.at[addr_v.at[j]], ss0).wait()
                return 0
            lax.fori_loop(0, nb, _deg_drain, 0)

        @pl.when(cid == 0)
        def _():
            _deg_run(nb0)

        @pl.when(cid == 1)
        def _():
            _deg_run(nb1)

        plsc.subcore_barrier()
        pltpu.sync_copy(s_sh.at[pl.ds(row0, rpt)],
                        deg_out.at[cid, pl.ds(row0, rpt)])


def _sc_aggregate(hc, srcb, addrb, z_hbm, nb0, nb1, with_deg):
    nchunk, _, _ = hc.shape
    rows_s = z_hbm.shape[0]
    mesh = plsc.VectorSubcoreMesh(core_axis_name="c", subcore_axis_name="s",
                                  num_cores=_NC, num_subcores=_NS)
    out_type = [jax.ShapeDtypeStruct((_NC, nchunk, rows_s, _CW), jnp.float32)]
    scratch = [
        pltpu.VMEM((nb0, _EB), jnp.int32),
        pltpu.VMEM((nb0, _EB), jnp.int32),
    ] + [pltpu.VMEM((_EB, _CW), jnp.float32)] * 4 + [
        pltpu.VMEM_SHARED((rows_s, _CW), jnp.float32),
    ] + [pltpu.SemaphoreType.DMA] * 8
    inputs = [hc, srcb, addrb, z_hbm]
    if with_deg:
        out_type.append(
            jax.ShapeDtypeStruct((_NC, rows_s, _CW), jnp.float32))
    body = functools.partial(_sc_agg_body, nchunk=nchunk, nb0=nb0, nb1=nb1,
                             rows_s=rows_s, with_deg=with_deg)
    return pl.kernel(
        body, out_type=out_type, mesh=mesh, scratch_types=scratch,
        compiler_params=pltpu.CompilerParams(use_tc_tiling_on_sc=False),
    )(*inputs)


def _layer_body(h_ref, s_ref, deg_ref, wroot_ref, wrel_ref, b_ref, out_ref,
                *, relu, nchunk):
    h = h_ref[...]
    dsum = deg_ref[0] + deg_ref[1]
    n0 = 1.0 / jnp.clip(dsum[0][:, 0:1], 1.0, None)
    n1 = 1.0 / jnp.clip(dsum[1][:, 0:1], 1.0, None)
    s0 = jnp.concatenate(
        [s_ref[0, c, 0] + s_ref[1, c, 0] for c in range(nchunk)], axis=1)
    s1 = jnp.concatenate(
        [s_ref[0, c, 1] + s_ref[1, c, 1] for c in range(nchunk)], axis=1)
    acc = jnp.dot(h, wroot_ref[...], preferred_element_type=jnp.float32)
    acc = acc + jnp.dot(s0 * n0, wrel_ref[0],
                        preferred_element_type=jnp.float32)
    acc = acc + jnp.dot(s1 * n1, wrel_ref[1],
                        preferred_element_type=jnp.float32)
    acc = acc + b_ref[...]
    out_ref[...] = jnp.maximum(acc, 0.0) if relu else acc


def _rgcn_dense(h, S5, deg4, W_root, W_rel, b, relu):
    n, f = h.shape
    o = W_rel.shape[2]
    nchunk = S5.shape[1]
    npr = S5.shape[3]
    bn = 1000
    grid = n // bn
    return pl.pallas_call(
        functools.partial(_layer_body, relu=relu, nchunk=nchunk),
        grid=(grid,),
        in_specs=[
            pl.BlockSpec((bn, f), lambda i: (i, 0)),
            pl.BlockSpec((2, nchunk, 2, bn, _CW), lambda i: (0, 0, 0, i, 0)),
            pl.BlockSpec((2, 2, bn, _CW), lambda i: (0, 0, i, 0)),
            pl.BlockSpec((f, o), lambda i: (0, 0)),
            pl.BlockSpec((2, f, o), lambda i: (0, 0, 0)),
            pl.BlockSpec((1, o), lambda i: (0, 0)),
        ],
        out_specs=pl.BlockSpec((bn, o), lambda i: (i, 0)),
        out_shape=jax.ShapeDtypeStruct((n, o), jnp.float32),
    )(h, S5, deg4, W_root, W_rel, b.reshape(1, o))


def _lstm_body(msg_ref, emb_ref, wih_ref, whh_ref, bg_ref, wfc_ref, bfc_ref,
               out_ref, xg_ref, *, L, B, HID, VOCAB):
    m = msg_ref[...]
    mx = jnp.max(m, axis=1, keepdims=True)
    col = lax.broadcasted_iota(jnp.int32, m.shape, 1)
    tok = jnp.min(jnp.where(m == mx, col, VOCAB), axis=1, keepdims=True)
    onehot = (col == tok).astype(jnp.float32)
    xe = jnp.dot(onehot, emb_ref[...], preferred_element_type=jnp.float32)
    xg = jnp.dot(xe, wih_ref[...], preferred_element_type=jnp.float32)
    xg = xg + bg_ref[...]
    xg_ref[...] = xg.reshape(L, B, 4 * HID)

    def step(l, carry):
        h, c = carry
        gates = xg_ref[l] + jnp.dot(h, whh_ref[...],
                                    preferred_element_type=jnp.float32)
        i = jax.nn.sigmoid(gates[:, :HID])
        f = jax.nn.sigmoid(gates[:, HID:2 * HID])
        g = jnp.tanh(gates[:, 2 * HID:3 * HID])
        o = jax.nn.sigmoid(gates[:, 3 * HID:])
        c = f * c + i * g
        h = o * jnp.tanh(c)
        return (h, c)

    h0 = jnp.zeros((B, HID), jnp.float32)
    hT, _ = lax.fori_loop(0, L, step, (h0, h0))
    out_ref[...] = jnp.dot(hT, wfc_ref[...],
                           preferred_element_type=jnp.float32) + bfc_ref[...]


def _msg_branch(message, emb_table, W_ih, W_hh, b_ih, b_hh, W_fc, b_fc):
    B, L, VOCAB = message.shape
    HID = W_hh.shape[1]
    EMB = emb_table.shape[1]
    msg_tm2d = jnp.swapaxes(message, 0, 1).reshape(L * B, VOCAB)
    bg = (b_ih + b_hh).reshape(1, 4 * HID)
    return pl.pallas_call(
        functools.partial(_lstm_body, L=L, B=B, HID=HID, VOCAB=VOCAB),
        out_shape=jax.ShapeDtypeStruct((B, EMB), jnp.float32),
        scratch_shapes=[pltpu.VMEM((L, B, 4 * HID), jnp.float32)],
    )(msg_tm2d, emb_table, W_ih.T, W_hh.T, bg, W_fc.T, b_fc.reshape(1, EMB))


def _logits_body(hid_ref, h_ref, out_ref):
    lg = lax.dot_general(hid_ref[...], h_ref[...], (((1,), (1,)), ((), ())),
                         preferred_element_type=jnp.float32)
    mx = jnp.max(lg, axis=1, keepdims=True)
    lse = jnp.log(jnp.sum(jnp.exp(lg - mx), axis=1, keepdims=True)) + mx
    out_ref[...] = lg - lse


def _logits(hidden, h):
    B = hidden.shape[0]
    n = h.shape[0]
    return pl.pallas_call(
        _logits_body,
        out_shape=jax.ShapeDtypeStruct((B, n), jnp.float32),
    )(hidden, h)


def _chunked(h):
    n, f = h.shape
    return jnp.transpose(h.reshape(n, f // _CW, _CW), (1, 0, 2))


def kernel(message, x, node_feats, edge_index, edge_type, nest_tensor,
           W_root1, W_rel1, b1, W_root2, W_rel2, b2, emb_table,
           W_ih, W_hh, b_ih, b_hh, W_fc, b_fc):
    src = edge_index[0]
    dst = edge_index[1]
    n = node_feats.shape[0]
    e = src.shape[0]
    nchunk = node_feats.shape[1] // _CW

    npr = -(-(n + 1) // 64) * 64
    rows_s = 2 * npr
    tb = -(-e // _EB)
    nb1 = max(4, (int(tb * 0.30) // (4 * _NS)) * 4)
    nb0 = -(-(tb - _NS * nb1) // (4 * _NS)) * 4
    epad = _NS * (nb0 + nb1) * _EB
    pad = epad - e
    srcb = jnp.concatenate(
        [src, jnp.zeros((pad,), jnp.int32)]).reshape(-1, _EB)
    addr = edge_type * npr + dst
    addrb = jnp.concatenate(
        [addr, jnp.full((pad,), n, jnp.int32)]).reshape(-1, _EB)
    z_hbm = jnp.zeros((rows_s, _CW), jnp.float32)

    S1, deg = _sc_aggregate(_chunked(node_feats), srcb, addrb, z_hbm,
                            nb0, nb1, with_deg=True)
    S1 = S1.reshape(_NC, nchunk, 2, npr, _CW)
    deg4 = deg.reshape(_NC, 2, npr, _CW)
    h1 = _rgcn_dense(node_feats, S1, deg4, W_root1, W_rel1, b1, relu=True)
    (S2,) = _sc_aggregate(_chunked(h1), srcb, addrb, z_hbm,
                          nb0, nb1, with_deg=False)
    S2 = S2.reshape(_NC, nchunk, 2, npr, _CW)
    h = _rgcn_dense(h1, S2, deg4, W_root2, W_rel2, b2, relu=False)

    hidden = _msg_branch(message, emb_table, W_ih, W_hh, b_ih, b_hh,
                         W_fc, b_fc)
    return _logits(hidden, h)


# speedup vs baseline: 1.0440x; 1.0007x over previous
import functools

import jax
import jax.numpy as jnp
from jax import lax
from jax.experimental import pallas as pl
from jax.experimental.pallas import tpu as pltpu
from jax.experimental.pallas import tpu_sc as plsc

_NC = 2
_NS = 16
_CW = 64
_EB = 128


def _sc_agg_body(hc_hbm, src_hbm, addr_hbm, z_hbm, *rest, nchunk, nb0, nb1,
                 rows_s, with_deg):
    if with_deg:
        (s_out, deg_out, src_v, addr_v, r0, r1, r2, r3, s_sh,
         sg0, sg1, sg2, sg3, ss0, ss1, ss2, ss3) = rest
    else:
        (s_out, src_v, addr_v, r0, r1, r2, r3, s_sh,
         sg0, sg1, sg2, sg3, ss0, ss1, ss2, ss3) = rest
        deg_out = None
    bufs = (r0, r1, r2, r3)
    sgs = (sg0, sg1, sg2, sg3)
    sss = (ss0, ss1, ss2, ss3)
    cid = lax.axis_index("c")
    sid = lax.axis_index("s")
    rpt = rows_s // _NS
    row0 = sid * rpt

    @pl.when(cid == 0)
    def _():
        pltpu.sync_copy(src_hbm.at[pl.ds(sid * nb0, nb0)], src_v)
        pltpu.sync_copy(addr_hbm.at[pl.ds(sid * nb0, nb0)], addr_v)

    @pl.when(cid == 1)
    def _():
        base = _NS * nb0 + sid * nb1
        pltpu.sync_copy(src_hbm.at[pl.ds(base, nb1)],
                        src_v.at[pl.ds(0, nb1)])
        pltpu.sync_copy(addr_hbm.at[pl.ds(base, nb1)],
                        addr_v.at[pl.ds(0, nb1)])

    def _zero_own():
        pltpu.sync_copy(z_hbm.at[pl.ds(row0, rpt)],
                        s_sh.at[pl.ds(row0, rpt)])

    for c in range(nchunk):
        with jax.named_scope(f"agg_zero{c}"):
            _zero_own()
            plsc.subcore_barrier()

        hcc = hc_hbm.at[c]

        def _fire_g(j, slot):
            pltpu.async_copy(hcc.at[src_v.at[j]], bufs[slot], sgs[slot])

        def _drain_g(j, slot):
            pltpu.make_async_copy(hcc.at[src_v.at[j]], bufs[slot],
                                  sgs[slot]).wait()

        def _fire_s(j, slot):
            pltpu.async_copy(bufs[slot], s_sh.at[addr_v.at[j]], sss[slot],
                             add=True)

        def _drain_s(j, slot):
            pltpu.make_async_copy(bufs[slot], s_sh.at[addr_v.at[j]],
                                  sss[slot]).wait()

        def _pipeline(nb):
            _fire_g(0, 0)
            _fire_g(1, 1)

            def _pipe(g, _):
                for u in range(4):
                    j = 4 * g + u
                    _drain_g(j, u)
                    _fire_s(j, u)

                    @pl.when(j + 2 < nb)
                    def _():
                        @pl.when(j >= 2)
                        def _():
                            _drain_s(j - 2, (u + 2) % 4)
                        _fire_g(j + 2, (u + 2) % 4)
                return 0

            lax.fori_loop(0, nb // 4, _pipe, 0)
            for k in range(nb - 4, nb):
                _drain_s(k, k % 4)

        with jax.named_scope(f"agg_edges{c}"):
            @pl.when(cid == 0)
            def _():
                _pipeline(nb0)

            @pl.when(cid == 1)
            def _():
                _pipeline(nb1)

            plsc.subcore_barrier()

        with jax.named_scope(f"agg_wb{c}"):
            pltpu.sync_copy(s_sh.at[pl.ds(row0, rpt)],
                            s_out.at[cid, c, pl.ds(row0, rpt)])
            plsc.subcore_barrier()

    if with_deg:
        _zero_own()

        def _fill(i, _):
            for k in range(_CW // 16):
                r0[i, pl.ds(16 * k, 16)] = jnp.ones((16,), jnp.float32)
            return 0
        lax.fori_loop(0, _EB, _fill, 0)
        plsc.subcore_barrier()

        def _deg_run(nb):
            def _deg_fire(j, _):
                pltpu.async_copy(r0, s_sh.at[addr_v.at[j]], ss0, add=True)
                return 0
            lax.fori_loop(0, nb, _deg_fire, 0)

            def _deg_drain(j, _):
                pltpu.make_async_copy(r0, s_sh.at[addr_v.at[j]], ss0).wait()
                return 0
            lax.fori_loop(0, nb, _deg_drain, 0)

        @pl.when(cid == 0)
        def _():
            _deg_run(nb0)

        @pl.when(cid == 1)
        def _():
            _deg_run(nb1)

        plsc.subcore_barrier()
        pltpu.sync_copy(s_sh.at[pl.ds(row0, rpt)],
                        deg_out.at[cid, pl.ds(row0, rpt)])


def _sc_aggregate(hc, srcb, addrb, z_hbm, nb0, nb1, with_deg):
    nchunk, _, _ = hc.shape
    rows_s = z_hbm.shape[0]
    mesh = plsc.VectorSubcoreMesh(core_axis_name="c", subcore_axis_name="s",
                                  num_cores=_NC, num_subcores=_NS)
    out_type = [jax.ShapeDtypeStruct((_NC, nchunk, rows_s, _CW), jnp.float32)]
    scratch = [
        pltpu.VMEM((nb0, _EB), jnp.int32),
        pltpu.VMEM((nb0, _EB), jnp.int32),
    ] + [pltpu.VMEM((_EB, _CW), jnp.float32)] * 4 + [
        pltpu.VMEM_SHARED((rows_s, _CW), jnp.float32),
    ] + [pltpu.SemaphoreType.DMA] * 8
    inputs = [hc, srcb, addrb, z_hbm]
    if with_deg:
        out_type.append(
            jax.ShapeDtypeStruct((_NC, rows_s, _CW), jnp.float32))
    body = functools.partial(_sc_agg_body, nchunk=nchunk, nb0=nb0, nb1=nb1,
                             rows_s=rows_s, with_deg=with_deg)
    return pl.kernel(
        body, out_type=out_type, mesh=mesh, scratch_types=scratch,
        compiler_params=pltpu.CompilerParams(use_tc_tiling_on_sc=False),
    )(*inputs)


def _layer_body(h_ref, s_ref, deg_ref, wroot_ref, wrel_ref, b_ref, out_ref,
                *, relu, nchunk):
    h = h_ref[...]
    dsum = deg_ref[0] + deg_ref[1]
    n0 = 1.0 / jnp.clip(dsum[0][:, 0:1], 1.0, None)
    n1 = 1.0 / jnp.clip(dsum[1][:, 0:1], 1.0, None)
    s0 = jnp.concatenate(
        [s_ref[0, c, 0] + s_ref[1, c, 0] for c in range(nchunk)], axis=1)
    s1 = jnp.concatenate(
        [s_ref[0, c, 1] + s_ref[1, c, 1] for c in range(nchunk)], axis=1)
    acc = jnp.dot(h, wroot_ref[...], preferred_element_type=jnp.float32)
    acc = acc + jnp.dot(s0 * n0, wrel_ref[0],
                        preferred_element_type=jnp.float32)
    acc = acc + jnp.dot(s1 * n1, wrel_ref[1],
                        preferred_element_type=jnp.float32)
    acc = acc + b_ref[...]
    out_ref[...] = jnp.maximum(acc, 0.0) if relu else acc


def _rgcn_dense(h, S5, deg4, W_root, W_rel, b, relu):
    n, f = h.shape
    o = W_rel.shape[2]
    nchunk = S5.shape[1]
    npr = S5.shape[3]
    bn = 1000
    grid = n // bn
    return pl.pallas_call(
        functools.partial(_layer_body, relu=relu, nchunk=nchunk),
        grid=(grid,),
        in_specs=[
            pl.BlockSpec((bn, f), lambda i: (i, 0)),
            pl.BlockSpec((2, nchunk, 2, bn, _CW), lambda i: (0, 0, 0, i, 0)),
            pl.BlockSpec((2, 2, bn, _CW), lambda i: (0, 0, i, 0)),
            pl.BlockSpec((f, o), lambda i: (0, 0)),
            pl.BlockSpec((2, f, o), lambda i: (0, 0, 0)),
            pl.BlockSpec((1, o), lambda i: (0, 0)),
        ],
        out_specs=pl.BlockSpec((bn, o), lambda i: (i, 0)),
        out_shape=jax.ShapeDtypeStruct((n, o), jnp.float32),
    )(h, S5, deg4, W_root, W_rel, b.reshape(1, o))


def _lstm_body(msg_ref, emb_ref, wih_ref, whh_ref, bg_ref, wfc_ref, bfc_ref,
               out_ref, xg_ref, *, L, B, HID, VOCAB):
    m = msg_ref[...]
    mx = jnp.max(m, axis=1, keepdims=True)
    col = lax.broadcasted_iota(jnp.int32, m.shape, 1)
    tok = jnp.min(jnp.where(m == mx, col, VOCAB), axis=1, keepdims=True)
    onehot = (col == tok).astype(jnp.float32)
    xe = jnp.dot(onehot, emb_ref[...], preferred_element_type=jnp.float32)
    xg = jnp.dot(xe, wih_ref[...], preferred_element_type=jnp.float32)
    xg = xg + bg_ref[...]
    xg_ref[...] = xg.reshape(L, B, 4 * HID)

    def step(l, carry):
        h, c = carry
        gates = xg_ref[l] + jnp.dot(h, whh_ref[...],
                                    preferred_element_type=jnp.float32)
        i = jax.nn.sigmoid(gates[:, :HID])
        f = jax.nn.sigmoid(gates[:, HID:2 * HID])
        g = jnp.tanh(gates[:, 2 * HID:3 * HID])
        o = jax.nn.sigmoid(gates[:, 3 * HID:])
        c = f * c + i * g
        h = o * jnp.tanh(c)
        return (h, c)

    h0 = jnp.zeros((B, HID), jnp.float32)
    hT, _ = lax.fori_loop(0, L, step, (h0, h0))
    out_ref[...] = jnp.dot(hT, wfc_ref[...],
                           preferred_element_type=jnp.float32) + bfc_ref[...]


def _msg_branch(message, emb_table, W_ih, W_hh, b_ih, b_hh, W_fc, b_fc):
    B, L, VOCAB = message.shape
    HID = W_hh.shape[1]
    EMB = emb_table.shape[1]
    msg_tm2d = jnp.swapaxes(message, 0, 1).reshape(L * B, VOCAB)
    bg = (b_ih + b_hh).reshape(1, 4 * HID)
    return pl.pallas_call(
        functools.partial(_lstm_body, L=L, B=B, HID=HID, VOCAB=VOCAB),
        out_shape=jax.ShapeDtypeStruct((B, EMB), jnp.float32),
        scratch_shapes=[pltpu.VMEM((L, B, 4 * HID), jnp.float32)],
    )(msg_tm2d, emb_table, W_ih.T, W_hh.T, bg, W_fc.T, b_fc.reshape(1, EMB))


def _logits_body(hid_ref, h_ref, out_ref):
    lg = lax.dot_general(hid_ref[...], h_ref[...], (((1,), (1,)), ((), ())),
                         preferred_element_type=jnp.float32)
    mx = jnp.max(lg, axis=1, keepdims=True)
    lse = jnp.log(jnp.sum(jnp.exp(lg - mx), axis=1, keepdims=True)) + mx
    out_ref[...] = lg - lse


def _logits(hidden, h):
    B = hidden.shape[0]
    n = h.shape[0]
    return pl.pallas_call(
        _logits_body,
        out_shape=jax.ShapeDtypeStruct((B, n), jnp.float32),
    )(hidden, h)


def _chunked(h):
    n, f = h.shape
    return jnp.transpose(h.reshape(n, f // _CW, _CW), (1, 0, 2))


def kernel(message, x, node_feats, edge_index, edge_type, nest_tensor,
           W_root1, W_rel1, b1, W_root2, W_rel2, b2, emb_table,
           W_ih, W_hh, b_ih, b_hh, W_fc, b_fc):
    src = edge_index[0]
    dst = edge_index[1]
    n = node_feats.shape[0]
    e = src.shape[0]
    nchunk = node_feats.shape[1] // _CW

    npr = -(-(n + 1) // 64) * 64
    rows_s = 2 * npr
    tb = -(-e // _EB)
    nb1 = max(4, (int(tb * 0.30) // (4 * _NS)) * 4)
    nb0 = -(-(tb - _NS * nb1) // (4 * _NS)) * 4
    epad = _NS * (nb0 + nb1) * _EB
    pad = epad - e
    srcb = jnp.concatenate(
        [src, jnp.zeros((pad,), jnp.int32)]).reshape(-1, _EB)
    addr = edge_type * npr + dst
    addrb = jnp.concatenate(
        [addr, jnp.full((pad,), n, jnp.int32)]).reshape(-1, _EB)
    z_hbm = jnp.zeros((rows_s, _CW), jnp.float32)

    S1, deg = _sc_aggregate(_chunked(node_feats), srcb, addrb, z_hbm,
                            nb0, nb1, with_deg=True)
    S1 = S1.reshape(_NC, nchunk, 2, npr, _CW)
    deg4 = deg.reshape(_NC, 2, npr, _CW)
    h1 = _rgcn_dense(node_feats, S1, deg4, W_root1, W_rel1, b1, relu=True)
    (S2,) = _sc_aggregate(_chunked(h1), srcb, addrb, z_hbm,
                          nb0, nb1, with_deg=False)
    S2 = S2.reshape(_NC, nchunk, 2, npr, _CW)
    h = _rgcn_dense(h1, S2, deg4, W_root2, W_rel2, b2, relu=False)

    hidden = _msg_branch(message, emb_table, W_ih, W_hh, b_ih, b_hh,
                         W_fc, b_fc)
    return _logits(hidden, h)
